# baseline (speedup 1.0000x reference)
import functools

import jax
import jax.numpy as jnp
from jax import lax
from jax.experimental import pallas as pl
from jax.experimental.pallas import tpu as pltpu
from jax.experimental.pallas import tpu_sc as plsc

VOCAB = 1000000
DIM = 64
BATCH = 16384

NUM_CORES = 2
NUM_SUBCORES = 16
LANES = 16
NUM_WORKERS = NUM_CORES * NUM_SUBCORES
BPW = BATCH // NUM_WORKERS
GROUPS = BPW // LANES
NTILES = VOCAB // 16


def _sc_body(x_hbm, t_hbm, tab_hbm, out_hbm,
             xi_v, ti_v, xa_v, ta_v, xr_v, tr_v, xc_v, tc_v, xt_v, tt_v, o_v, sem):
    wid = lax.axis_index("s") * NUM_CORES + lax.axis_index("c")
    base = pl.multiple_of(wid * BPW, BPW)

    pltpu.sync_copy(x_hbm.at[pl.ds(base, BPW)], xi_v)
    pltpu.sync_copy(t_hbm.at[pl.ds(base, BPW)], ti_v)

    lane = lax.iota(jnp.int32, LANES)

    for g in range(GROUPS):
        sl = pl.ds(g * LANES, LANES)
        xv = xi_v[sl]
        tv = ti_v[sl]
        xa_v[sl] = xv >> 4
        ta_v[sl] = tv >> 4
        xr_v[sl] = (xv >> 1) & 7
        tr_v[sl] = (tv >> 1) & 7
        xc_v[sl] = (xv & 1) * DIM
        tc_v[sl] = (tv & 1) * DIM

    def group_body(g, carry):
        sl = pl.ds(pl.multiple_of(g * LANES, LANES), LANES)
        cx = pltpu.async_copy(tab_hbm.at[xa_v.at[sl]], xt_v, sem)
        ct = pltpu.async_copy(tab_hbm.at[ta_v.at[sl]], tt_v, sem)
        cx.wait()
        ct.wait()
        rx = xr_v[sl]
        rt = tr_v[sl]
        cx0 = xc_v[sl]
        ct0 = tc_v[sl]
        acc = jnp.zeros((LANES,), jnp.float32)
        for d in range(DIM):
            xa = plsc.load_gather(xt_v, [lane, rx, cx0 + d])
            ta = plsc.load_gather(tt_v, [lane, rt, ct0 + d])
            acc = acc + xa * ta
        o_v[sl] = acc
        return carry

    lax.fori_loop(0, GROUPS, group_body, 0)

    pltpu.sync_copy(o_v, out_hbm.at[pl.ds(base, BPW)])


@jax.jit
def kernel(x, t, table):
    mesh = plsc.VectorSubcoreMesh(
        core_axis_name="c", subcore_axis_name="s",
        num_cores=NUM_CORES, num_subcores=NUM_SUBCORES)
    f = pl.kernel(
        _sc_body,
        out_type=jax.ShapeDtypeStruct((BATCH,), jnp.float32),
        mesh=mesh,
        compiler_params=pltpu.CompilerParams(needs_layout_passes=False),
        scratch_types=[
            pltpu.VMEM((BPW,), jnp.int32),
            pltpu.VMEM((BPW,), jnp.int32),
            pltpu.VMEM((BPW,), jnp.int32),
            pltpu.VMEM((BPW,), jnp.int32),
            pltpu.VMEM((BPW,), jnp.int32),
            pltpu.VMEM((BPW,), jnp.int32),
            pltpu.VMEM((BPW,), jnp.int32),
            pltpu.VMEM((BPW,), jnp.int32),
            pltpu.VMEM((LANES, 8, 128), jnp.float32),
            pltpu.VMEM((LANES, 8, 128), jnp.float32),
            pltpu.VMEM((BPW,), jnp.float32),
            pltpu.SemaphoreType.DMA,
        ],
    )
    tab = table.reshape(NTILES, 8, 128)
    return f(x.astype(jnp.int32), t.astype(jnp.int32), tab)


# speedup vs baseline: 1.5817x; 1.5817x over previous
import jax
import jax.numpy as jnp
from jax import lax
from jax.experimental import pallas as pl
from jax.experimental.pallas import tpu as pltpu
from jax.experimental.pallas import tpu_sc as plsc

VOCAB = 1000000
DIM = 64
BATCH = 16384

NUM_CORES = 2
NUM_SUBCORES = 16
LANES = 16
NUM_WORKERS = NUM_CORES * NUM_SUBCORES
BPW = BATCH // NUM_WORKERS
NBUF = 4


def _extract(vec_ref, p, lane):
    g = pl.multiple_of((p // LANES) * LANES, LANES)
    v16 = vec_ref[pl.ds(g, LANES)]
    return jnp.sum(jnp.where(lane == p % LANES, v16, 0))


def _sc_body(x_hbm, t_hbm, tab_hbm, out_hbm, xi_v, ti_v, xt_v, tt_v, o_v, sem):
    wid = lax.axis_index("s") * NUM_CORES + lax.axis_index("c")
    base = pl.multiple_of(wid * BPW, BPW)

    pltpu.sync_copy(x_hbm.at[pl.ds(base, BPW)], xi_v)
    pltpu.sync_copy(t_hbm.at[pl.ds(base, BPW)], ti_v)

    lane = lax.iota(jnp.int32, LANES)

    def issue(p, buf):
        vx = _extract(xi_v, p, lane)
        vt = _extract(ti_v, p, lane)
        ax = pl.multiple_of((vx >> 3) * 8, 8)
        at = pl.multiple_of((vt >> 3) * 8, 8)
        pltpu.async_copy(tab_hbm.at[pl.ds(ax, 8), :], xt_v.at[buf], sem)
        pltpu.async_copy(tab_hbm.at[pl.ds(at, 8), :], tt_v.at[buf], sem)
        return vx, vt

    for b in range(NBUF):
        issue(b, b)

    def outer(g, acc):
        for b in range(NBUF):
            p = g * NBUF + b
            pltpu.make_async_copy(
                tab_hbm.at[pl.ds(0, 8), :], xt_v.at[b], sem).wait()
            pltpu.make_async_copy(
                tab_hbm.at[pl.ds(0, 8), :], tt_v.at[b], sem).wait()
            vx = _extract(xi_v, p, lane)
            vt = _extract(ti_v, p, lane)
            rx = vx & 7
            rt = vt & 7
            s = jnp.zeros((LANES,), jnp.float32)
            for k in range(4):
                sl = pl.ds(k * LANES, LANES)
                s = s + xt_v[b, rx, sl] * tt_v[b, rt, sl]
            acc = acc + jnp.where(lane == p % LANES, jnp.sum(s), 0.0)
            pn = jnp.minimum(p + NBUF, BPW - 1)
            vx2 = _extract(xi_v, pn, lane)
            vt2 = _extract(ti_v, pn, lane)
            ax = pl.multiple_of((vx2 >> 3) * 8, 8)
            at = pl.multiple_of((vt2 >> 3) * 8, 8)
            pltpu.async_copy(tab_hbm.at[pl.ds(ax, 8), :], xt_v.at[b], sem)
            pltpu.async_copy(tab_hbm.at[pl.ds(at, 8), :], tt_v.at[b], sem)

        @pl.when((g % (LANES // NBUF)) == (LANES // NBUF - 1))
        def _():
            go = pl.multiple_of((g * NBUF // LANES) * LANES, LANES)
            o_v[pl.ds(go, LANES)] = acc

        return jnp.where(
            (g % (LANES // NBUF)) == (LANES // NBUF - 1),
            jnp.zeros((LANES,), jnp.float32), acc)

    lax.fori_loop(0, BPW // NBUF, outer, jnp.zeros((LANES,), jnp.float32))

    for b in range(NBUF):
        pltpu.make_async_copy(
            tab_hbm.at[pl.ds(0, 8), :], xt_v.at[b], sem).wait()
        pltpu.make_async_copy(
            tab_hbm.at[pl.ds(0, 8), :], tt_v.at[b], sem).wait()

    pltpu.sync_copy(o_v, out_hbm.at[pl.ds(base, BPW)])


@jax.jit
def kernel(x, t, table):
    mesh = plsc.VectorSubcoreMesh(
        core_axis_name="c", subcore_axis_name="s",
        num_cores=NUM_CORES, num_subcores=NUM_SUBCORES)
    f = pl.kernel(
        _sc_body,
        out_type=jax.ShapeDtypeStruct((BATCH,), jnp.float32),
        mesh=mesh,
        compiler_params=pltpu.CompilerParams(needs_layout_passes=False),
        scratch_types=[
            pltpu.VMEM((BPW,), jnp.int32),
            pltpu.VMEM((BPW,), jnp.int32),
            pltpu.VMEM((NBUF, 8, DIM), jnp.float32),
            pltpu.VMEM((NBUF, 8, DIM), jnp.float32),
            pltpu.VMEM((BPW,), jnp.float32),
            pltpu.SemaphoreType.DMA,
        ],
    )
    return f(x.astype(jnp.int32), t.astype(jnp.int32), table)


# speedup vs baseline: 1.7200x; 1.0875x over previous
import jax
import jax.numpy as jnp
from jax import lax
from jax.experimental import pallas as pl
from jax.experimental.pallas import tpu as pltpu
from jax.experimental.pallas import tpu_sc as plsc

VOCAB = 1000000
DIM = 64
BATCH = 16384

NUM_CORES = 2
NUM_SUBCORES = 16
LANES = 16
NUM_WORKERS = NUM_CORES * NUM_SUBCORES
BPW = BATCH // NUM_WORKERS
NBUF = 8


def _extract(vec_ref, p, lane):
    g = pl.multiple_of((p // LANES) * LANES, LANES)
    v16 = vec_ref[pl.ds(g, LANES)]
    return jnp.sum(jnp.where(lane == p % LANES, v16, 0))


def _sc_body(x_hbm, t_hbm, tab_hbm, out_hbm, xi_v, ti_v, xt_v, tt_v, o_v, sem):
    wid = lax.axis_index("s") * NUM_CORES + lax.axis_index("c")
    base = pl.multiple_of(wid * BPW, BPW)

    pltpu.sync_copy(x_hbm.at[pl.ds(base, BPW)], xi_v)
    pltpu.sync_copy(t_hbm.at[pl.ds(base, BPW)], ti_v)

    lane = lax.iota(jnp.int32, LANES)

    def issue(p, buf):
        vx = _extract(xi_v, p, lane)
        vt = _extract(ti_v, p, lane)
        ax = pl.multiple_of((vx >> 3) * 8, 8)
        at = pl.multiple_of((vt >> 3) * 8, 8)
        pltpu.async_copy(tab_hbm.at[pl.ds(ax, 8), :], xt_v.at[buf], sem)
        pltpu.async_copy(tab_hbm.at[pl.ds(at, 8), :], tt_v.at[buf], sem)
        return vx, vt

    for b in range(NBUF):
        issue(b, b)

    def outer(g, acc):
        for b in range(NBUF):
            p = g * NBUF + b
            pltpu.make_async_copy(
                tab_hbm.at[pl.ds(0, 8), :], xt_v.at[b], sem).wait()
            pltpu.make_async_copy(
                tab_hbm.at[pl.ds(0, 8), :], tt_v.at[b], sem).wait()
            vx = _extract(xi_v, p, lane)
            vt = _extract(ti_v, p, lane)
            rx = vx & 7
            rt = vt & 7
            s = jnp.zeros((LANES,), jnp.float32)
            for k in range(4):
                sl = pl.ds(k * LANES, LANES)
                s = s + xt_v[b, rx, sl] * tt_v[b, rt, sl]
            acc = acc + jnp.where(lane == p % LANES, jnp.sum(s), 0.0)
            pn = jnp.minimum(p + NBUF, BPW - 1)
            vx2 = _extract(xi_v, pn, lane)
            vt2 = _extract(ti_v, pn, lane)
            ax = pl.multiple_of((vx2 >> 3) * 8, 8)
            at = pl.multiple_of((vt2 >> 3) * 8, 8)
            pltpu.async_copy(tab_hbm.at[pl.ds(ax, 8), :], xt_v.at[b], sem)
            pltpu.async_copy(tab_hbm.at[pl.ds(at, 8), :], tt_v.at[b], sem)

        @pl.when((g % (LANES // NBUF)) == (LANES // NBUF - 1))
        def _():
            go = pl.multiple_of((g * NBUF // LANES) * LANES, LANES)
            o_v[pl.ds(go, LANES)] = acc

        return jnp.where(
            (g % (LANES // NBUF)) == (LANES // NBUF - 1),
            jnp.zeros((LANES,), jnp.float32), acc)

    lax.fori_loop(0, BPW // NBUF, outer, jnp.zeros((LANES,), jnp.float32))

    for b in range(NBUF):
        pltpu.make_async_copy(
            tab_hbm.at[pl.ds(0, 8), :], xt_v.at[b], sem).wait()
        pltpu.make_async_copy(
            tab_hbm.at[pl.ds(0, 8), :], tt_v.at[b], sem).wait()

    pltpu.sync_copy(o_v, out_hbm.at[pl.ds(base, BPW)])


@jax.jit
def kernel(x, t, table):
    mesh = plsc.VectorSubcoreMesh(
        core_axis_name="c", subcore_axis_name="s",
        num_cores=NUM_CORES, num_subcores=NUM_SUBCORES)
    f = pl.kernel(
        _sc_body,
        out_type=jax.ShapeDtypeStruct((BATCH,), jnp.float32),
        mesh=mesh,
        compiler_params=pltpu.CompilerParams(needs_layout_passes=False),
        scratch_types=[
            pltpu.VMEM((BPW,), jnp.int32),
            pltpu.VMEM((BPW,), jnp.int32),
            pltpu.VMEM((NBUF, 8, DIM), jnp.float32),
            pltpu.VMEM((NBUF, 8, DIM), jnp.float32),
            pltpu.VMEM((BPW,), jnp.float32),
            pltpu.SemaphoreType.DMA,
        ],
    )
    return f(x.astype(jnp.int32), t.astype(jnp.int32), table)


# speedup vs baseline: 1.7730x; 1.0308x over previous
import jax
import jax.numpy as jnp
from jax import lax
from jax.experimental import pallas as pl
from jax.experimental.pallas import tpu as pltpu
from jax.experimental.pallas import tpu_sc as plsc

VOCAB = 1000000
DIM = 64
BATCH = 16384

NUM_CORES = 2
NUM_SUBCORES = 16
LANES = 16
NUM_WORKERS = NUM_CORES * NUM_SUBCORES
BPW = BATCH // NUM_WORKERS
NBUF = 16


def _extract(vec_ref, p, lane):
    g = pl.multiple_of((p // LANES) * LANES, LANES)
    v16 = vec_ref[pl.ds(g, LANES)]
    return jnp.sum(jnp.where(lane == p % LANES, v16, 0))


def _sc_body(x_hbm, t_hbm, tab_hbm, out_hbm, xi_v, ti_v, xt_v, tt_v, o_v, sem):
    wid = lax.axis_index("s") * NUM_CORES + lax.axis_index("c")
    base = pl.multiple_of(wid * BPW, BPW)

    pltpu.sync_copy(x_hbm.at[pl.ds(base, BPW)], xi_v)
    pltpu.sync_copy(t_hbm.at[pl.ds(base, BPW)], ti_v)

    lane = lax.iota(jnp.int32, LANES)

    def issue(p, buf):
        vx = _extract(xi_v, p, lane)
        vt = _extract(ti_v, p, lane)
        ax = pl.multiple_of((vx >> 3) * 8, 8)
        at = pl.multiple_of((vt >> 3) * 8, 8)
        pltpu.async_copy(tab_hbm.at[pl.ds(ax, 8), :], xt_v.at[buf], sem)
        pltpu.async_copy(tab_hbm.at[pl.ds(at, 8), :], tt_v.at[buf], sem)
        return vx, vt

    for b in range(NBUF):
        issue(b, b)

    def outer(g, acc):
        for b in range(NBUF):
            p = g * NBUF + b
            pltpu.make_async_copy(
                tab_hbm.at[pl.ds(0, 8), :], xt_v.at[b], sem).wait()
            pltpu.make_async_copy(
                tab_hbm.at[pl.ds(0, 8), :], tt_v.at[b], sem).wait()
            vx = _extract(xi_v, p, lane)
            vt = _extract(ti_v, p, lane)
            rx = vx & 7
            rt = vt & 7
            s = jnp.zeros((LANES,), jnp.float32)
            for k in range(4):
                sl = pl.ds(k * LANES, LANES)
                s = s + xt_v[b, rx, sl] * tt_v[b, rt, sl]
            acc = acc + jnp.where(lane == p % LANES, jnp.sum(s), 0.0)
            pn = jnp.minimum(p + NBUF, BPW - 1)
            vx2 = _extract(xi_v, pn, lane)
            vt2 = _extract(ti_v, pn, lane)
            ax = pl.multiple_of((vx2 >> 3) * 8, 8)
            at = pl.multiple_of((vt2 >> 3) * 8, 8)
            pltpu.async_copy(tab_hbm.at[pl.ds(ax, 8), :], xt_v.at[b], sem)
            pltpu.async_copy(tab_hbm.at[pl.ds(at, 8), :], tt_v.at[b], sem)

        @pl.when((g % (LANES // NBUF)) == (LANES // NBUF - 1))
        def _():
            go = pl.multiple_of((g * NBUF // LANES) * LANES, LANES)
            o_v[pl.ds(go, LANES)] = acc

        return jnp.where(
            (g % (LANES // NBUF)) == (LANES // NBUF - 1),
            jnp.zeros((LANES,), jnp.float32), acc)

    lax.fori_loop(0, BPW // NBUF, outer, jnp.zeros((LANES,), jnp.float32))

    for b in range(NBUF):
        pltpu.make_async_copy(
            tab_hbm.at[pl.ds(0, 8), :], xt_v.at[b], sem).wait()
        pltpu.make_async_copy(
            tab_hbm.at[pl.ds(0, 8), :], tt_v.at[b], sem).wait()

    pltpu.sync_copy(o_v, out_hbm.at[pl.ds(base, BPW)])


@jax.jit
def kernel(x, t, table):
    mesh = plsc.VectorSubcoreMesh(
        core_axis_name="c", subcore_axis_name="s",
        num_cores=NUM_CORES, num_subcores=NUM_SUBCORES)
    f = pl.kernel(
        _sc_body,
        out_type=jax.ShapeDtypeStruct((BATCH,), jnp.float32),
        mesh=mesh,
        compiler_params=pltpu.CompilerParams(needs_layout_passes=False),
        scratch_types=[
            pltpu.VMEM((BPW,), jnp.int32),
            pltpu.VMEM((BPW,), jnp.int32),
            pltpu.VMEM((NBUF, 8, DIM), jnp.float32),
            pltpu.VMEM((NBUF, 8, DIM), jnp.float32),
            pltpu.VMEM((BPW,), jnp.float32),
            pltpu.SemaphoreType.DMA,
        ],
    )
    return f(x.astype(jnp.int32), t.astype(jnp.int32), table)
